# baseline (speedup 1.0000x reference)
import functools

import jax
import jax.numpy as jnp
from jax import lax
from jax.experimental import pallas as pl
from jax.experimental.pallas import tpu as pltpu
from jax.experimental.pallas import tpu_sc as plsc

N = 10000
E = 320000
D = 128
H = 64
NC = 2
NS = 16
L = 16

CHUNK = 128
NCHT = 158
TOTCH = NS * NCHT
EPAD = TOTCH * CHUNK - E
RPT = 624
REM = N - NS * RPT
NSUB = 208


def _sc_propagate2():
    mesh = plsc.VectorSubcoreMesh(core_axis_name="c", subcore_axis_name="s")

    @functools.partial(
        pl.kernel,
        mesh=mesh,
        compiler_params=pltpu.CompilerParams(
            needs_layout_passes=False, use_tc_tiling_on_sc=False),
        out_type=(
            jax.ShapeDtypeStruct((2 * N, H), jnp.float32),
            jax.ShapeDtypeStruct((2 * N, H), jnp.float32),
        ),
        scratch_types=[
            pltpu.VMEM_SHARED((N + 8, H), jnp.float32),
            pltpu.VMEM_SHARED((N + 8, H), jnp.float32),
            pltpu.VMEM_SHARED((N + 8,), jnp.float32),
            pltpu.VMEM((3, CHUNK), jnp.int32),
            pltpu.VMEM((3, CHUNK), jnp.int32),
            pltpu.VMEM((CHUNK, H), jnp.float32),
            pltpu.VMEM((CHUNK, H), jnp.float32),
            pltpu.VMEM((CHUNK,), jnp.float32),
            pltpu.VMEM((NSUB, H), jnp.float32),
            pltpu.VMEM((NSUB,), jnp.float32),
            pltpu.SemaphoreType.DMA,
            pltpu.SemaphoreType.DMA,
        ],
    )
    def k(xp, packed, z2, z1, o1, t1p, t2p,
          agg1, agg2, deg, idxb0, idxb1, rows0, rows1,
          ones, wb, degb, semg, semi):
        c = lax.axis_index("c")
        s = lax.axis_index("s")
        r0 = s * RPT
        off = c * N

        pltpu.sync_copy(z2, wb)
        pltpu.sync_copy(z1, degb)

        def zero_slice(base, nrows):
            pltpu.sync_copy(wb.at[pl.ds(0, nrows)], agg1.at[pl.ds(base, nrows)])
            pltpu.sync_copy(wb.at[pl.ds(0, nrows)], agg2.at[pl.ds(base, nrows)])
            pltpu.sync_copy(degb.at[pl.ds(0, nrows)], deg.at[pl.ds(base, nrows)])

        for u in range(RPT // NSUB):
            zero_slice(r0 + u * NSUB, NSUB)

        @pl.when(s == NS - 1)
        def _():
            zero_slice(NS * RPT, REM)

        pltpu.sync_copy(o1, ones)
        plsc.subcore_barrier()

        def edge_loop(gsrc, sel, acc, with_deg):
            def idx_start(i, ib):
                cid = lax.min(s + NS * i, TOTCH - 1)
                pltpu.async_copy(packed.at[cid], ib, semi)

            def idx_wait(ib):
                pltpu.make_async_copy(packed.at[0], ib, semi).wait()

            def g_start(ib, rb):
                pltpu.async_copy(gsrc.at[ib.at[sel]], rb, semg)

            def g_wait(ib, rb):
                pltpu.make_async_copy(gsrc.at[ib.at[sel]], rb, semg).wait()

            def scatter(rb, ib):
                pltpu.sync_copy(rb, acc.at[ib.at[2]], add=True)
                if with_deg:
                    pltpu.sync_copy(ones, deg.at[ib.at[2]], add=True)

            idx_start(0, idxb0)
            idx_wait(idxb0)
            g_start(idxb0, rows0)
            idx_start(1, idxb1)

            def pair(j, _):
                a = 2 * j
                idx_wait(idxb1)
                g_wait(idxb0, rows0)
                g_start(idxb1, rows1)
                scatter(rows0, idxb0)
                idx_start(a + 2, idxb0)
                g_wait(idxb1, rows1)
                idx_wait(idxb0)
                g_start(idxb0, rows0)
                scatter(rows1, idxb1)
                idx_start(a + 3, idxb1)
                return 0

            lax.fori_loop(0, NCHT // 2, pair, 0)
            idx_wait(idxb1)
            g_wait(idxb0, rows0)

        edge_loop(xp, c, agg1, True)
        plsc.subcore_barrier()

        def normalize_slice(acc, out_hbm, writeback, base, nrows):
            pltpu.sync_copy(acc.at[pl.ds(base, nrows)], wb.at[pl.ds(0, nrows)])
            pltpu.sync_copy(deg.at[pl.ds(base, nrows)], degb.at[pl.ds(0, nrows)])

            def nbody(r, _):
                idx = jnp.full((L,), r, jnp.int32)
                dv = plsc.load_gather(degb, [idx])
                sc = 1.0 / jnp.maximum(dv, 1.0)
                for cc in range(H // L):
                    wb[r, pl.ds(cc * L, L)] = wb[r, pl.ds(cc * L, L)] * sc
                return 0

            lax.fori_loop(0, nrows, nbody, 0)
            if writeback:
                pltpu.sync_copy(wb.at[pl.ds(0, nrows)], acc.at[pl.ds(base, nrows)])
            pltpu.sync_copy(wb.at[pl.ds(0, nrows)],
                            out_hbm.at[pl.ds(off + base, nrows)])

        def normalize(acc, out_hbm, writeback):
            for u in range(RPT // NSUB):
                normalize_slice(acc, out_hbm, writeback, r0 + u * NSUB, NSUB)

            @pl.when(s == NS - 1)
            def _():
                normalize_slice(acc, out_hbm, writeback, NS * RPT, REM)

        normalize(agg1, t1p, False)
        plsc.subcore_barrier()

        edge_loop(t1p, c, agg2, False)
        plsc.subcore_barrier()

        normalize(agg2, t2p, False)

    return k


_sc_prop = _sc_propagate2()


def _linear_tc(x, t1p, t2p, wt, b2):
    R = 1000
    nb = N // R

    def body(x_ref, t1lo, t1hi, t2lo, t2hi, w_ref, b_ref, o_ref):
        w = w_ref[...]
        a = w[0:D] - w[2 * D:3 * D]
        wb1 = w[D:2 * D]
        wc = w[2 * D:3 * D] * 2.0
        acc = jnp.dot(x_ref[...], a, preferred_element_type=jnp.float32)
        acc += jnp.dot(t1lo[...], wb1[:H], preferred_element_type=jnp.float32)
        acc += jnp.dot(t1hi[...], wb1[H:], preferred_element_type=jnp.float32)
        acc += jnp.dot(t2lo[...], wc[:H], preferred_element_type=jnp.float32)
        acc += jnp.dot(t2hi[...], wc[H:], preferred_element_type=jnp.float32)
        o_ref[...] = acc + b_ref[...]

    return pl.pallas_call(
        body,
        grid=(nb,),
        in_specs=[
            pl.BlockSpec((R, D), lambda i: (i, 0)),
            pl.BlockSpec((R, H), lambda i: (i, 0)),
            pl.BlockSpec((R, H), lambda i: (i + nb, 0)),
            pl.BlockSpec((R, H), lambda i: (i, 0)),
            pl.BlockSpec((R, H), lambda i: (i + nb, 0)),
            pl.BlockSpec((3 * D, D), lambda i: (0, 0)),
            pl.BlockSpec((1, D), lambda i: (0, 0)),
        ],
        out_specs=pl.BlockSpec((R, D), lambda i: (i, 0)),
        out_shape=jax.ShapeDtypeStruct((N, D), jnp.float32),
    )(x, t1p, t1p, t2p, t2p, wt, b2)


def kernel(x, edge_index, W, b):
    xp = jnp.concatenate([x[:, :H], x[:, H:]], axis=0)
    src = jnp.concatenate([edge_index[0], jnp.zeros((EPAD,), jnp.int32)])
    dst = jnp.concatenate([edge_index[1], jnp.full((EPAD,), N, jnp.int32)])
    packed = jnp.stack([src.reshape(TOTCH, CHUNK),
                        (src + N).reshape(TOTCH, CHUNK),
                        dst.reshape(TOTCH, CHUNK)], axis=1)
    z2 = jnp.zeros((NSUB, H), jnp.float32)
    z1 = jnp.zeros((NSUB,), jnp.float32)
    o1 = jnp.ones((CHUNK,), jnp.float32)
    t1p, t2p = _sc_prop(xp, packed, z2, z1, o1)
    return _linear_tc(x, t1p, t2p, W.T, b.reshape(1, D))


# speedup vs baseline: 1.1566x; 1.1566x over previous
import functools

import jax
import jax.numpy as jnp
from jax import lax
from jax.experimental import pallas as pl
from jax.experimental.pallas import tpu as pltpu
from jax.experimental.pallas import tpu_sc as plsc

N = 10000
E = 320000
D = 128
H = 64
NC = 2
NS = 16
L = 16

CHUNK = 128
NCHT = 159
TOTCH = NS * NCHT
EPAD = TOTCH * CHUNK - E
RPT = 624
REM = N - NS * RPT
NSUB = 208


def _sc_propagate2():
    mesh = plsc.VectorSubcoreMesh(core_axis_name="c", subcore_axis_name="s")

    @functools.partial(
        pl.kernel,
        mesh=mesh,
        compiler_params=pltpu.CompilerParams(
            needs_layout_passes=False, use_tc_tiling_on_sc=False),
        out_type=(
            jax.ShapeDtypeStruct((2 * N, H), jnp.float32),
            jax.ShapeDtypeStruct((2 * N, H), jnp.float32),
        ),
        scratch_types=[
            pltpu.VMEM_SHARED((N + 8, H), jnp.float32),
            pltpu.VMEM_SHARED((N + 8, H), jnp.float32),
            pltpu.VMEM_SHARED((N + 8,), jnp.float32),
            pltpu.VMEM((3, CHUNK), jnp.int32),
            pltpu.VMEM((3, CHUNK), jnp.int32),
            pltpu.VMEM((3, CHUNK), jnp.int32),
            pltpu.VMEM((CHUNK, H), jnp.float32),
            pltpu.VMEM((CHUNK, H), jnp.float32),
            pltpu.VMEM((CHUNK, H), jnp.float32),
            pltpu.VMEM((CHUNK,), jnp.int32),
            pltpu.VMEM((CHUNK,), jnp.int32),
            pltpu.VMEM((CHUNK,), jnp.int32),
            pltpu.VMEM((CHUNK,), jnp.float32),
            pltpu.VMEM((NSUB, H), jnp.float32),
            pltpu.VMEM((NSUB,), jnp.float32),
            pltpu.SemaphoreType.DMA,
            pltpu.SemaphoreType.DMA,
            pltpu.SemaphoreType.DMA,
        ],
    )
    def k(xp, packed, z2, z1, o1, t1p, t2p,
          agg1, agg2, deg, idxb0, idxb1, idxb2, rows0, rows1, rows2,
          dstc0, dstc1, dstc2, ones, wb, degb, semg, semi, sems):
        c = lax.axis_index("c")
        s = lax.axis_index("s")
        r0 = s * RPT
        off = c * N

        pltpu.sync_copy(z2, wb)
        pltpu.sync_copy(z1, degb)

        def zero_slice(base, nrows):
            pltpu.sync_copy(wb.at[pl.ds(0, nrows)], agg1.at[pl.ds(base, nrows)])
            pltpu.sync_copy(wb.at[pl.ds(0, nrows)], agg2.at[pl.ds(base, nrows)])
            pltpu.sync_copy(degb.at[pl.ds(0, nrows)], deg.at[pl.ds(base, nrows)])

        for u in range(RPT // NSUB):
            zero_slice(r0 + u * NSUB, NSUB)

        @pl.when(s == NS - 1)
        def _():
            zero_slice(NS * RPT, REM)

        pltpu.sync_copy(o1, ones)
        plsc.subcore_barrier()

        def edge_loop(gsrc, sel, acc, with_deg):
            idxb = (idxb0, idxb1, idxb2)
            rows = (rows0, rows1, rows2)
            dstc = (dstc0, dstc1, dstc2)

            def idx_start(i, k):
                cid = lax.min(s + NS * i, TOTCH - 1)
                pltpu.async_copy(packed.at[cid], idxb[k], semi)

            def idx_wait():
                pltpu.make_async_copy(packed.at[0], idxb0, semi).wait()

            def g_start(k):
                pltpu.async_copy(gsrc.at[idxb[k].at[sel]], rows[k], semg)

            def g_wait(k):
                pltpu.make_async_copy(gsrc.at[idxb[k].at[sel]], rows[k],
                                      semg).wait()

            def dst_copy(k):
                for j in range(CHUNK // L):
                    dstc[k][pl.ds(j * L, L)] = idxb[k][2, pl.ds(j * L, L)]

            def sc_start(k):
                pltpu.async_copy(rows[k], acc.at[dstc[k]], sems, add=True)
                if with_deg:
                    pltpu.async_copy(ones, deg.at[dstc[k]], sems, add=True)

            def sc_drain(k):
                pltpu.make_async_copy(rows[k], acc.at[dstc[k]], sems).wait()
                if with_deg:
                    pltpu.make_async_copy(ones, deg.at[dstc[k]], sems).wait()

            def step(i, k, drain):
                g_wait(k)
                dst_copy(k)
                if drain:
                    sc_drain((k + 1) % 3)
                sc_start(k)
                idx_start(i + 3, k)
                idx_wait()
                g_start((k + 1) % 3)

            idx_start(0, 0)
            idx_start(1, 1)
            idx_start(2, 2)
            idx_wait()
            g_start(0)
            step(0, 0, False)
            step(1, 1, False)
            step(2, 2, True)

            def triple(j, _):
                i0 = 3 * j
                step(i0, 0, True)
                step(i0 + 1, 1, True)
                step(i0 + 2, 2, True)
                return 0

            lax.fori_loop(1, NCHT // 3, triple, 0)
            g_wait(0)
            sc_drain(1)
            sc_drain(2)
            idx_wait()
            idx_wait()

        edge_loop(xp, c, agg1, True)
        plsc.subcore_barrier()

        def normalize_slice(acc, out_hbm, writeback, base, nrows):
            pltpu.sync_copy(acc.at[pl.ds(base, nrows)], wb.at[pl.ds(0, nrows)])
            pltpu.sync_copy(deg.at[pl.ds(base, nrows)], degb.at[pl.ds(0, nrows)])

            def nbody(r, _):
                idx = jnp.full((L,), r, jnp.int32)
                dv = plsc.load_gather(degb, [idx])
                sc = 1.0 / jnp.maximum(dv, 1.0)
                for cc in range(H // L):
                    wb[r, pl.ds(cc * L, L)] = wb[r, pl.ds(cc * L, L)] * sc
                return 0

            lax.fori_loop(0, nrows, nbody, 0)
            if writeback:
                pltpu.sync_copy(wb.at[pl.ds(0, nrows)], acc.at[pl.ds(base, nrows)])
            pltpu.sync_copy(wb.at[pl.ds(0, nrows)],
                            out_hbm.at[pl.ds(off + base, nrows)])

        def normalize(acc, out_hbm, writeback):
            for u in range(RPT // NSUB):
                normalize_slice(acc, out_hbm, writeback, r0 + u * NSUB, NSUB)

            @pl.when(s == NS - 1)
            def _():
                normalize_slice(acc, out_hbm, writeback, NS * RPT, REM)

        normalize(agg1, t1p, True)
        plsc.subcore_barrier()

        edge_loop(agg1, 0, agg2, False)
        plsc.subcore_barrier()

        normalize(agg2, t2p, False)

    return k


_sc_prop = _sc_propagate2()


def _linear_tc(x, t1p, t2p, wt, b2):
    R = 1000
    nb = N // R

    def body(x_ref, t1lo, t1hi, t2lo, t2hi, w_ref, b_ref, o_ref):
        w = w_ref[...]
        a = w[0:D] - w[2 * D:3 * D]
        wb1 = w[D:2 * D]
        wc = w[2 * D:3 * D] * 2.0
        acc = jnp.dot(x_ref[...], a, preferred_element_type=jnp.float32)
        acc += jnp.dot(t1lo[...], wb1[:H], preferred_element_type=jnp.float32)
        acc += jnp.dot(t1hi[...], wb1[H:], preferred_element_type=jnp.float32)
        acc += jnp.dot(t2lo[...], wc[:H], preferred_element_type=jnp.float32)
        acc += jnp.dot(t2hi[...], wc[H:], preferred_element_type=jnp.float32)
        o_ref[...] = acc + b_ref[...]

    return pl.pallas_call(
        body,
        grid=(nb,),
        in_specs=[
            pl.BlockSpec((R, D), lambda i: (i, 0)),
            pl.BlockSpec((R, H), lambda i: (i, 0)),
            pl.BlockSpec((R, H), lambda i: (i + nb, 0)),
            pl.BlockSpec((R, H), lambda i: (i, 0)),
            pl.BlockSpec((R, H), lambda i: (i + nb, 0)),
            pl.BlockSpec((3 * D, D), lambda i: (0, 0)),
            pl.BlockSpec((1, D), lambda i: (0, 0)),
        ],
        out_specs=pl.BlockSpec((R, D), lambda i: (i, 0)),
        out_shape=jax.ShapeDtypeStruct((N, D), jnp.float32),
    )(x, t1p, t1p, t2p, t2p, wt, b2)


def kernel(x, edge_index, W, b):
    xp = jnp.concatenate([x[:, :H], x[:, H:]], axis=0)
    src = jnp.concatenate([edge_index[0], jnp.zeros((EPAD,), jnp.int32)])
    dst = jnp.concatenate([edge_index[1], jnp.full((EPAD,), N, jnp.int32)])
    packed = jnp.stack([src.reshape(TOTCH, CHUNK),
                        (src + N).reshape(TOTCH, CHUNK),
                        dst.reshape(TOTCH, CHUNK)], axis=1)
    z2 = jnp.zeros((NSUB, H), jnp.float32)
    z1 = jnp.zeros((NSUB,), jnp.float32)
    o1 = jnp.ones((CHUNK,), jnp.float32)
    t1p, t2p = _sc_prop(xp, packed, z2, z1, o1)
    return _linear_tc(x, t1p, t2p, W.T, b.reshape(1, D))


# speedup vs baseline: 1.1567x; 1.0001x over previous
import functools

import jax
import jax.numpy as jnp
from jax import lax
from jax.experimental import pallas as pl
from jax.experimental.pallas import tpu as pltpu
from jax.experimental.pallas import tpu_sc as plsc

N = 10000
E = 320000
D = 128
H = 64
NC = 2
NS = 16
L = 16

CHUNK = 128
NCHT = 159
TOTCH = NS * NCHT
EPAD = TOTCH * CHUNK - E
RPT = 624
REM = N - NS * RPT
NSUB = 208


def _sc_propagate2():
    mesh = plsc.VectorSubcoreMesh(core_axis_name="c", subcore_axis_name="s")

    @functools.partial(
        pl.kernel,
        mesh=mesh,
        compiler_params=pltpu.CompilerParams(
            needs_layout_passes=False, use_tc_tiling_on_sc=False),
        out_type=(
            jax.ShapeDtypeStruct((2 * N, H), jnp.float32),
            jax.ShapeDtypeStruct((2 * N, H), jnp.float32),
        ),
        scratch_types=[
            pltpu.VMEM_SHARED((N + 8, H), jnp.float32),
            pltpu.VMEM_SHARED((N + 8, H), jnp.float32),
            pltpu.VMEM_SHARED((N + 8,), jnp.float32),
            pltpu.VMEM((3, CHUNK), jnp.int32),
            pltpu.VMEM((3, CHUNK), jnp.int32),
            pltpu.VMEM((3, CHUNK), jnp.int32),
            pltpu.VMEM((CHUNK, H), jnp.float32),
            pltpu.VMEM((CHUNK, H), jnp.float32),
            pltpu.VMEM((CHUNK, H), jnp.float32),
            pltpu.VMEM((CHUNK,), jnp.int32),
            pltpu.VMEM((CHUNK,), jnp.int32),
            pltpu.VMEM((CHUNK,), jnp.int32),
            pltpu.VMEM((CHUNK,), jnp.float32),
            pltpu.VMEM((NSUB, H), jnp.float32),
            pltpu.VMEM((NSUB,), jnp.float32),
            pltpu.SemaphoreType.DMA,
            pltpu.SemaphoreType.DMA,
            pltpu.SemaphoreType.DMA,
        ],
    )
    def k(xp, packed, z2, z1, o1, t1p, t2p,
          agg1, agg2, deg, idxb0, idxb1, idxb2, rows0, rows1, rows2,
          dstc0, dstc1, dstc2, ones, wb, degb, semg, semi, sems):
        c = lax.axis_index("c")
        s = lax.axis_index("s")
        r0 = s * RPT
        off = c * N

        pltpu.sync_copy(z2, wb)
        pltpu.sync_copy(z1, degb)

        def zero_slice(base, nrows):
            pltpu.sync_copy(wb.at[pl.ds(0, nrows)], agg1.at[pl.ds(base, nrows)])
            pltpu.sync_copy(wb.at[pl.ds(0, nrows)], agg2.at[pl.ds(base, nrows)])
            pltpu.sync_copy(degb.at[pl.ds(0, nrows)], deg.at[pl.ds(base, nrows)])

        for u in range(RPT // NSUB):
            zero_slice(r0 + u * NSUB, NSUB)

        @pl.when(s == NS - 1)
        def _():
            zero_slice(NS * RPT, REM)

        pltpu.sync_copy(o1, ones)
        plsc.subcore_barrier()

        def edge_loop(gsrc, sel, acc, with_deg):
            idxb = (idxb0, idxb1, idxb2)
            rows = (rows0, rows1, rows2)
            dstc = (dstc0, dstc1, dstc2)

            def idx_start(i, k):
                cid = lax.min(s + NS * i, TOTCH - 1)
                pltpu.async_copy(packed.at[cid], idxb[k], semi)

            def idx_wait():
                pltpu.make_async_copy(packed.at[0], idxb0, semi).wait()

            def g_start(k):
                pltpu.async_copy(gsrc.at[idxb[k].at[sel]], rows[k], semg)

            def g_wait(k):
                pltpu.make_async_copy(gsrc.at[idxb[k].at[sel]], rows[k],
                                      semg).wait()

            def dst_copy(k):
                for j in range(CHUNK // L):
                    dstc[k][pl.ds(j * L, L)] = idxb[k][2, pl.ds(j * L, L)]

            def sc_start(k):
                pltpu.async_copy(rows[k], acc.at[dstc[k]], sems, add=True)
                if with_deg:
                    pltpu.async_copy(ones, deg.at[dstc[k]], sems, add=True)

            def sc_drain(k):
                pltpu.make_async_copy(rows[k], acc.at[dstc[k]], sems).wait()
                if with_deg:
                    pltpu.make_async_copy(ones, deg.at[dstc[k]], sems).wait()

            def step(i, k, drain):
                g_wait(k)
                dst_copy(k)
                if drain:
                    sc_drain((k + 1) % 3)
                idx_wait()
                g_start((k + 1) % 3)
                sc_start(k)
                idx_start(i + 3, k)

            idx_start(0, 0)
            idx_start(1, 1)
            idx_start(2, 2)
            idx_wait()
            g_start(0)
            step(0, 0, False)
            step(1, 1, False)
            step(2, 2, True)

            def triple(j, _):
                i0 = 3 * j
                step(i0, 0, True)
                step(i0 + 1, 1, True)
                step(i0 + 2, 2, True)
                return 0

            lax.fori_loop(1, NCHT // 3, triple, 0)
            g_wait(0)
            sc_drain(1)
            sc_drain(2)
            idx_wait()
            idx_wait()

        edge_loop(xp, c, agg1, True)
        plsc.subcore_barrier()

        def normalize_slice(acc, out_hbm, writeback, base, nrows):
            pltpu.sync_copy(acc.at[pl.ds(base, nrows)], wb.at[pl.ds(0, nrows)])
            pltpu.sync_copy(deg.at[pl.ds(base, nrows)], degb.at[pl.ds(0, nrows)])

            def nbody(r, _):
                idx = jnp.full((L,), r, jnp.int32)
                dv = plsc.load_gather(degb, [idx])
                sc = 1.0 / jnp.maximum(dv, 1.0)
                for cc in range(H // L):
                    wb[r, pl.ds(cc * L, L)] = wb[r, pl.ds(cc * L, L)] * sc
                return 0

            lax.fori_loop(0, nrows, nbody, 0)
            if writeback:
                pltpu.sync_copy(wb.at[pl.ds(0, nrows)], acc.at[pl.ds(base, nrows)])
            pltpu.sync_copy(wb.at[pl.ds(0, nrows)],
                            out_hbm.at[pl.ds(off + base, nrows)])

        def normalize(acc, out_hbm, writeback):
            for u in range(RPT // NSUB):
                normalize_slice(acc, out_hbm, writeback, r0 + u * NSUB, NSUB)

            @pl.when(s == NS - 1)
            def _():
                normalize_slice(acc, out_hbm, writeback, NS * RPT, REM)

        normalize(agg1, t1p, True)
        plsc.subcore_barrier()

        edge_loop(agg1, 0, agg2, False)
        plsc.subcore_barrier()

        normalize(agg2, t2p, False)

    return k


_sc_prop = _sc_propagate2()


def _linear_tc(x, t1p, t2p, wt, b2):
    R = 1000
    nb = N // R

    def body(x_ref, t1lo, t1hi, t2lo, t2hi, w_ref, b_ref, o_ref):
        w = w_ref[...]
        a = w[0:D] - w[2 * D:3 * D]
        wb1 = w[D:2 * D]
        wc = w[2 * D:3 * D] * 2.0
        acc = jnp.dot(x_ref[...], a, preferred_element_type=jnp.float32)
        acc += jnp.dot(t1lo[...], wb1[:H], preferred_element_type=jnp.float32)
        acc += jnp.dot(t1hi[...], wb1[H:], preferred_element_type=jnp.float32)
        acc += jnp.dot(t2lo[...], wc[:H], preferred_element_type=jnp.float32)
        acc += jnp.dot(t2hi[...], wc[H:], preferred_element_type=jnp.float32)
        o_ref[...] = acc + b_ref[...]

    return pl.pallas_call(
        body,
        grid=(nb,),
        in_specs=[
            pl.BlockSpec((R, D), lambda i: (i, 0)),
            pl.BlockSpec((R, H), lambda i: (i, 0)),
            pl.BlockSpec((R, H), lambda i: (i + nb, 0)),
            pl.BlockSpec((R, H), lambda i: (i, 0)),
            pl.BlockSpec((R, H), lambda i: (i + nb, 0)),
            pl.BlockSpec((3 * D, D), lambda i: (0, 0)),
            pl.BlockSpec((1, D), lambda i: (0, 0)),
        ],
        out_specs=pl.BlockSpec((R, D), lambda i: (i, 0)),
        out_shape=jax.ShapeDtypeStruct((N, D), jnp.float32),
    )(x, t1p, t1p, t2p, t2p, wt, b2)


def kernel(x, edge_index, W, b):
    xp = jnp.concatenate([x[:, :H], x[:, H:]], axis=0)
    src = jnp.concatenate([edge_index[0], jnp.zeros((EPAD,), jnp.int32)])
    dst = jnp.concatenate([edge_index[1], jnp.full((EPAD,), N, jnp.int32)])
    packed = jnp.stack([src.reshape(TOTCH, CHUNK),
                        (src + N).reshape(TOTCH, CHUNK),
                        dst.reshape(TOTCH, CHUNK)], axis=1)
    z2 = jnp.zeros((NSUB, H), jnp.float32)
    z1 = jnp.zeros((NSUB,), jnp.float32)
    o1 = jnp.ones((CHUNK,), jnp.float32)
    t1p, t2p = _sc_prop(xp, packed, z2, z1, o1)
    return _linear_tc(x, t1p, t2p, W.T, b.reshape(1, D))
